# baseline (speedup 1.0000x reference)
import dataclasses

import jax
import jax.numpy as jnp
from jax import lax
from jax.experimental import pallas as pl
from jax.experimental.pallas import tpu as pltpu
from jax.experimental.pallas import tpu_sc as plsc

_NUM_CORES = 2
_NUM_SUBCORES = 16
_NUM_WORKERS = _NUM_CORES * _NUM_SUBCORES
_CHUNK = 128
_LANES = 16


def kernel(inputs, lookup_table):
    batch, seq = inputs.shape
    vocab, dim = lookup_table.shape
    n = batch * seq
    rows_per_tile = 128 // dim
    n_chunks = n // _CHUNK
    chunks_per_worker = n_chunks // _NUM_WORKERS
    chunks_per_col = batch // _CHUNK
    assert n % (_NUM_WORKERS * _CHUNK) == 0 and batch % _CHUNK == 0
    assert chunks_per_worker % 2 == 0

    mesh = plsc.VectorSubcoreMesh(core_axis_name="c", subcore_axis_name="s")
    idx = inputs.T.reshape(n).astype(jnp.int32)

    cp = pltpu.CompilerParams()
    if "needs_layout_passes" in pltpu.CompilerParams.__dataclass_fields__:
        cp = dataclasses.replace(cp, needs_layout_passes=False)

    cols_per_block = 128
    full_blocks = vocab // cols_per_block
    tail_cols = vocab - full_blocks * cols_per_block
    whole = full_blocks // _NUM_WORKERS
    extra = full_blocks % _NUM_WORKERS
    assert whole % 2 == 0 and tail_cols % 8 == 0
    nat = lookup_table.T
    if tail_cols:
        tail16 = lookup_table[vocab - tail_cols :, :]
    else:
        tail16 = jnp.zeros((8, dim), jnp.float32)

    @pl.kernel(
        out_type=jax.ShapeDtypeStruct((vocab // rows_per_tile, 128), jnp.float32),
        mesh=mesh,
        compiler_params=cp,
        scratch_types=[
            pltpu.VMEM((dim, cols_per_block), jnp.float32),
            pltpu.VMEM((dim, cols_per_block), jnp.float32),
            pltpu.VMEM((cols_per_block // rows_per_tile, 128), jnp.float32),
            pltpu.VMEM((cols_per_block // rows_per_tile, 128), jnp.float32),
            pltpu.VMEM((max(tail_cols, 8), dim), jnp.float32),
            pltpu.SemaphoreType.DMA,
            pltpu.SemaphoreType.DMA,
        ],
    )
    def compact_kernel(
        nat_hbm, tail_hbm, out_hbm, in_a, in_b, cp_a, cp_b, tv, sem_a, sem_b
    ):
        wid = lax.axis_index("s") * _NUM_CORES + lax.axis_index("c")
        iota = lax.iota(jnp.int32, _LANES)
        out_rows = cols_per_block // rows_per_tile

        def start(blk, buf, sem):
            c0 = pl.multiple_of(blk * cols_per_block, 128)
            return pltpu.async_copy(
                nat_hbm.at[:, pl.ds(c0, cols_per_block)], buf, sem
            )

        def transpose(buf, cbuf, ncols):
            for r in range(ncols // rows_per_tile):
                for a in range(rows_per_tile):
                    src_col = rows_per_tile * r + a
                    for h in range(dim // _LANES):
                        cbuf[r, pl.ds(a * dim + h * _LANES, _LANES)] = (
                            plsc.load_gather(
                                buf, [h * _LANES + iota, src_col + iota * 0]
                            )
                        )

        def drain(blk, buf, cbuf, sem):
            pltpu.make_async_copy(
                nat_hbm.at[:, pl.ds(0, cols_per_block)], buf, sem
            ).wait()
            transpose(buf, cbuf, cols_per_block)
            c0 = pl.multiple_of(blk * out_rows, 8)
            pltpu.sync_copy(cbuf, out_hbm.at[pl.ds(c0, out_rows)])

        start(wid, in_a, sem_a)

        @pl.loop(0, whole // 2)
        def _(k):
            blk_a = wid + 2 * k * _NUM_WORKERS
            blk_b = blk_a + _NUM_WORKERS
            start(blk_b, in_b, sem_b)
            drain(blk_a, in_a, cp_a, sem_a)

            @pl.when(k + 1 < whole // 2)
            def _():
                start(blk_a + 2 * _NUM_WORKERS, in_a, sem_a)

            @pl.when((k + 1 == whole // 2) & (wid < extra))
            def _():
                start(wid + whole * _NUM_WORKERS, in_a, sem_a)

            drain(blk_b, in_b, cp_b, sem_b)

        @pl.when(wid < extra)
        def _():
            drain(wid + whole * _NUM_WORKERS, in_a, cp_a, sem_a)

        if tail_cols:
            @pl.when(wid == extra)
            def _():
                pltpu.sync_copy(tail_hbm, tv)
                for i in range(tail_cols):
                    r = i // rows_per_tile
                    c = (i % rows_per_tile) * dim
                    for h in range(dim // _LANES):
                        cp_b[r, pl.ds(c + h * _LANES, _LANES)] = tv[
                            i, pl.ds(h * _LANES, _LANES)
                        ]
                pltpu.sync_copy(
                    cp_b.at[pl.ds(0, tail_cols // rows_per_tile)],
                    out_hbm.at[
                        pl.ds(
                            full_blocks * out_rows, tail_cols // rows_per_tile
                        )
                    ],
                )

    table4 = compact_kernel(nat, tail16)

    @pl.kernel(
        out_type=jax.ShapeDtypeStruct((seq * dim, batch), jnp.float32),
        mesh=mesh,
        compiler_params=cp,
        scratch_types=[
            pltpu.VMEM((chunks_per_worker * _CHUNK,), jnp.int32),
            pltpu.VMEM((_CHUNK,), jnp.int32),
            pltpu.VMEM((_CHUNK,), jnp.int32),
            pltpu.VMEM((_CHUNK, 128), jnp.float32),
            pltpu.VMEM((_CHUNK, 128), jnp.float32),
            pltpu.VMEM((dim, _CHUNK), jnp.float32),
            pltpu.VMEM((dim, _CHUNK), jnp.float32),
            pltpu.SemaphoreType.DMA,
            pltpu.SemaphoreType.DMA,
            pltpu.SemaphoreType.DMA,
            pltpu.SemaphoreType.DMA,
        ],
    )
    def gather_kernel(
        table_hbm,
        idx_hbm,
        out_hbm,
        idx_all,
        r_a,
        r_b,
        rows_a,
        rows_b,
        trans_a,
        trans_b,
        sem_ga,
        sem_gb,
        sem_oa,
        sem_ob,
    ):
        wid = lax.axis_index("s") * _NUM_CORES + lax.axis_index("c")
        first = wid * chunks_per_worker
        iota = lax.iota(jnp.int32, _LANES)
        groups = _CHUNK // _LANES

        base0 = pl.multiple_of(first * _CHUNK, _CHUNK)
        pltpu.sync_copy(
            idx_hbm.at[pl.ds(base0, chunks_per_worker * _CHUNK)], idx_all
        )

        def fetch(lc, r_v, rows_v, sem_g):
            off = lc * _CHUNK
            for g in range(groups):
                v = idx_all[pl.ds(off + g * _LANES, _LANES)]
                r_v[pl.ds(g * _LANES, _LANES)] = lax.shift_right_logical(v, 2)
            return pltpu.async_copy(table_hbm.at[r_v], rows_v, sem_g)

        def select_store(lc, rows_v, trans_v, sem_o):
            off = lc * _CHUNK
            for g in range(groups):
                v = idx_all[pl.ds(off + g * _LANES, _LANES)]
                col = (v & (rows_per_tile - 1)) * dim
                row16 = g * _LANES + iota
                for e in range(dim):
                    val = plsc.load_gather(rows_v, [row16, col])
                    trans_v[e, pl.ds(g * _LANES, _LANES)] = val
                    if e + 1 < dim:
                        col = col + 1
            ch = first + lc
            t = ch // chunks_per_col
            b0 = (ch % chunks_per_col) * _CHUNK
            r0 = pl.multiple_of(t * dim, 8)
            return pltpu.async_copy(
                trans_v, out_hbm.at[pl.ds(r0, dim), pl.ds(b0, _CHUNK)], sem_o
            )

        fetch(0, r_a, rows_a, sem_ga)

        @pl.loop(0, chunks_per_worker // 2)
        def _(k):
            lca = 2 * k
            lcb = lca + 1
            fetch(lcb, r_b, rows_b, sem_gb)
            pltpu.make_async_copy(table_hbm.at[pl.ds(0, _CHUNK)], rows_a, sem_ga).wait()

            @pl.when(k > 0)
            def _():
                pltpu.make_async_copy(
                    trans_a, out_hbm.at[pl.ds(0, dim), pl.ds(0, _CHUNK)], sem_oa
                ).wait()

            select_store(lca, rows_a, trans_a, sem_oa)

            @pl.when(k + 1 < chunks_per_worker // 2)
            def _():
                fetch(lca + 2, r_a, rows_a, sem_ga)

            pltpu.make_async_copy(table_hbm.at[pl.ds(0, _CHUNK)], rows_b, sem_gb).wait()

            @pl.when(k > 0)
            def _():
                pltpu.make_async_copy(
                    trans_b, out_hbm.at[pl.ds(0, dim), pl.ds(0, _CHUNK)], sem_ob
                ).wait()

            select_store(lcb, rows_b, trans_b, sem_ob)

        pltpu.make_async_copy(
            trans_a, out_hbm.at[pl.ds(0, dim), pl.ds(0, _CHUNK)], sem_oa
        ).wait()
        pltpu.make_async_copy(
            trans_b, out_hbm.at[pl.ds(0, dim), pl.ds(0, _CHUNK)], sem_ob
        ).wait()

    out = gather_kernel(table4, idx)
    return out.reshape(seq, dim, batch).transpose(2, 0, 1)


# speedup vs baseline: 1.3492x; 1.3492x over previous
import dataclasses

import jax
import jax.numpy as jnp
from jax import lax
from jax.experimental import pallas as pl
from jax.experimental.pallas import tpu as pltpu
from jax.experimental.pallas import tpu_sc as plsc

_NUM_CORES = 2
_NUM_SUBCORES = 16
_NUM_WORKERS = _NUM_CORES * _NUM_SUBCORES
_CHUNK = 256
_STREAM = 128
_LANES = 16


def kernel(inputs, lookup_table):
    batch, seq = inputs.shape
    vocab, dim = lookup_table.shape
    n = batch * seq
    rows_per_tile = 128 // dim
    n_chunks = n // _CHUNK
    chunks_per_worker = n_chunks // _NUM_WORKERS
    chunks_per_col = batch // _CHUNK
    assert n % (_NUM_WORKERS * _CHUNK) == 0 and batch % _CHUNK == 0
    assert chunks_per_worker % 2 == 0

    mesh = plsc.VectorSubcoreMesh(core_axis_name="c", subcore_axis_name="s")
    idx = inputs.T.reshape(n).astype(jnp.int32)
    table4 = lookup_table.reshape(vocab // rows_per_tile, 128)

    cp = pltpu.CompilerParams()
    if "needs_layout_passes" in pltpu.CompilerParams.__dataclass_fields__:
        cp = dataclasses.replace(cp, needs_layout_passes=False)

    @pl.kernel(
        out_type=jax.ShapeDtypeStruct((seq * dim, batch), jnp.float32),
        mesh=mesh,
        compiler_params=cp,
        scratch_types=[
            pltpu.VMEM((chunks_per_worker * _CHUNK,), jnp.int32),
            pltpu.VMEM((_CHUNK,), jnp.int32),
            pltpu.VMEM((_CHUNK,), jnp.int32),
            pltpu.VMEM((_CHUNK, 128), jnp.float32),
            pltpu.VMEM((_CHUNK, 128), jnp.float32),
            pltpu.VMEM((dim, _CHUNK), jnp.float32),
            pltpu.VMEM((dim, _CHUNK), jnp.float32),
            pltpu.SemaphoreType.DMA,
            pltpu.SemaphoreType.DMA,
            pltpu.SemaphoreType.DMA,
            pltpu.SemaphoreType.DMA,
        ],
    )
    def gather_kernel(
        table_hbm,
        idx_hbm,
        out_hbm,
        idx_all,
        r_a,
        r_b,
        rows_a,
        rows_b,
        trans_a,
        trans_b,
        sem_ga,
        sem_gb,
        sem_oa,
        sem_ob,
    ):
        wid = lax.axis_index("s") * _NUM_CORES + lax.axis_index("c")
        first = wid * chunks_per_worker
        iota = lax.iota(jnp.int32, _LANES)
        groups = _CHUNK // _LANES

        base0 = pl.multiple_of(first * _CHUNK, _CHUNK)
        pltpu.sync_copy(
            idx_hbm.at[pl.ds(base0, chunks_per_worker * _CHUNK)], idx_all
        )

        def fetch(lc, r_v, rows_v, sem_g):
            off = lc * _CHUNK
            for g in range(groups):
                v = idx_all[pl.ds(off + g * _LANES, _LANES)]
                r_v[pl.ds(g * _LANES, _LANES)] = lax.shift_right_logical(v, 2)
            for s in range(_CHUNK // _STREAM):
                pltpu.async_copy(
                    table_hbm.at[r_v.at[pl.ds(s * _STREAM, _STREAM)]],
                    rows_v.at[pl.ds(s * _STREAM, _STREAM)],
                    sem_g,
                )

        def select_store(lc, rows_v, trans_v, sem_o):
            off = lc * _CHUNK
            for g in range(groups):
                v = idx_all[pl.ds(off + g * _LANES, _LANES)]
                col = (v & (rows_per_tile - 1)) * dim
                row16 = g * _LANES + iota
                for e in range(dim):
                    val = plsc.load_gather(rows_v, [row16, col])
                    trans_v[e, pl.ds(g * _LANES, _LANES)] = val
                    if e + 1 < dim:
                        col = col + 1
            ch = first + lc
            t = ch // chunks_per_col
            b0 = (ch % chunks_per_col) * _CHUNK
            r0 = pl.multiple_of(t * dim, 8)
            return pltpu.async_copy(
                trans_v, out_hbm.at[pl.ds(r0, dim), pl.ds(b0, _CHUNK)], sem_o
            )

        fetch(0, r_a, rows_a, sem_ga)

        @pl.loop(0, chunks_per_worker // 2)
        def _(k):
            lca = 2 * k
            lcb = lca + 1
            fetch(lcb, r_b, rows_b, sem_gb)
            pltpu.make_async_copy(
                table_hbm.at[pl.ds(0, _CHUNK)], rows_a, sem_ga
            ).wait()

            @pl.when(k > 0)
            def _():
                pltpu.make_async_copy(
                    trans_a, out_hbm.at[pl.ds(0, dim), pl.ds(0, _CHUNK)], sem_oa
                ).wait()

            select_store(lca, rows_a, trans_a, sem_oa)

            @pl.when(k + 1 < chunks_per_worker // 2)
            def _():
                fetch(lca + 2, r_a, rows_a, sem_ga)

            pltpu.make_async_copy(table_hbm.at[pl.ds(0, _CHUNK)], rows_b, sem_gb).wait()

            @pl.when(k > 0)
            def _():
                pltpu.make_async_copy(
                    trans_b, out_hbm.at[pl.ds(0, dim), pl.ds(0, _CHUNK)], sem_ob
                ).wait()

            select_store(lcb, rows_b, trans_b, sem_ob)

        pltpu.make_async_copy(
            trans_a, out_hbm.at[pl.ds(0, dim), pl.ds(0, _CHUNK)], sem_oa
        ).wait()
        pltpu.make_async_copy(
            trans_b, out_hbm.at[pl.ds(0, dim), pl.ds(0, _CHUNK)], sem_ob
        ).wait()

    out = gather_kernel(table4, idx)
    return out.reshape(seq, dim, batch).transpose(2, 0, 1)


# speedup vs baseline: 1.3646x; 1.0114x over previous
import dataclasses

import jax
import jax.numpy as jnp
from jax import lax
from jax.experimental import pallas as pl
from jax.experimental.pallas import tpu as pltpu
from jax.experimental.pallas import tpu_sc as plsc

_NUM_CORES = 2
_NUM_SUBCORES = 16
_NUM_WORKERS = _NUM_CORES * _NUM_SUBCORES
_CHUNK = 128
_LANES = 16


def kernel(inputs, lookup_table):
    batch, seq = inputs.shape
    vocab, dim = lookup_table.shape
    n = batch * seq
    rows_per_tile = 128 // dim
    n_chunks = n // _CHUNK
    chunks_per_worker = n_chunks // _NUM_WORKERS
    chunks_per_col = batch // _CHUNK
    assert n % (_NUM_WORKERS * _CHUNK) == 0 and batch % _CHUNK == 0
    assert chunks_per_worker % 2 == 0

    mesh = plsc.VectorSubcoreMesh(core_axis_name="c", subcore_axis_name="s")
    idx = inputs.T.reshape(n).astype(jnp.int32)
    table4 = lookup_table.reshape(vocab // rows_per_tile, 128)

    cp = pltpu.CompilerParams()
    if "needs_layout_passes" in pltpu.CompilerParams.__dataclass_fields__:
        cp = dataclasses.replace(cp, needs_layout_passes=False)

    @pl.kernel(
        out_type=jax.ShapeDtypeStruct((seq * dim, batch), jnp.float32),
        mesh=mesh,
        compiler_params=cp,
        scratch_types=[
            pltpu.VMEM((chunks_per_worker * _CHUNK,), jnp.int32),
            pltpu.VMEM((_CHUNK,), jnp.int32),
            pltpu.VMEM((_CHUNK,), jnp.int32),
            pltpu.VMEM((_CHUNK, 128), jnp.float32),
            pltpu.VMEM((_CHUNK, 128), jnp.float32),
            pltpu.VMEM((dim, _CHUNK), jnp.float32),
            pltpu.VMEM((dim, _CHUNK), jnp.float32),
            pltpu.SemaphoreType.DMA,
            pltpu.SemaphoreType.DMA,
            pltpu.SemaphoreType.DMA,
            pltpu.SemaphoreType.DMA,
        ],
    )
    def gather_kernel(
        table_hbm,
        idx_hbm,
        out_hbm,
        idx_all,
        r_a,
        r_b,
        rows_a,
        rows_b,
        trans_a,
        trans_b,
        sem_ga,
        sem_gb,
        sem_oa,
        sem_ob,
    ):
        wid = lax.axis_index("s") * _NUM_CORES + lax.axis_index("c")
        first = wid * chunks_per_worker
        iota = lax.iota(jnp.int32, _LANES)
        groups = _CHUNK // _LANES

        base0 = pl.multiple_of(first * _CHUNK, _CHUNK)
        pltpu.sync_copy(
            idx_hbm.at[pl.ds(base0, chunks_per_worker * _CHUNK)], idx_all
        )

        def fetch(lc, r_v, rows_v, sem_g):
            off = lc * _CHUNK
            for g in range(groups):
                v = idx_all[pl.ds(off + g * _LANES, _LANES)]
                r_v[pl.ds(g * _LANES, _LANES)] = lax.shift_right_logical(v, 2)
            return pltpu.async_copy(table_hbm.at[r_v], rows_v, sem_g)

        def select_store(lc, rows_v, trans_v, sem_o):
            off = lc * _CHUNK
            cols = []
            rows16 = []
            for g in range(groups):
                v = idx_all[pl.ds(off + g * _LANES, _LANES)]
                cols.append((v & (rows_per_tile - 1)) * dim)
                rows16.append(g * _LANES + iota)
            for e in range(dim):
                for g in range(groups):
                    val = plsc.load_gather(rows_v, [rows16[g], cols[g]])
                    trans_v[e, pl.ds(g * _LANES, _LANES)] = val
                    if e + 1 < dim:
                        cols[g] = cols[g] + 1
            ch = first + lc
            t = ch // chunks_per_col
            b0 = (ch % chunks_per_col) * _CHUNK
            r0 = pl.multiple_of(t * dim, 8)
            return pltpu.async_copy(
                trans_v, out_hbm.at[pl.ds(r0, dim), pl.ds(b0, _CHUNK)], sem_o
            )

        fetch(0, r_a, rows_a, sem_ga)

        @pl.loop(0, chunks_per_worker // 2)
        def _(k):
            lca = 2 * k
            lcb = lca + 1
            fetch(lcb, r_b, rows_b, sem_gb)
            pltpu.make_async_copy(table_hbm.at[pl.ds(0, _CHUNK)], rows_a, sem_ga).wait()

            @pl.when(k > 0)
            def _():
                pltpu.make_async_copy(
                    trans_a, out_hbm.at[pl.ds(0, dim), pl.ds(0, _CHUNK)], sem_oa
                ).wait()

            select_store(lca, rows_a, trans_a, sem_oa)

            @pl.when(k + 1 < chunks_per_worker // 2)
            def _():
                fetch(lca + 2, r_a, rows_a, sem_ga)

            pltpu.make_async_copy(table_hbm.at[pl.ds(0, _CHUNK)], rows_b, sem_gb).wait()

            @pl.when(k > 0)
            def _():
                pltpu.make_async_copy(
                    trans_b, out_hbm.at[pl.ds(0, dim), pl.ds(0, _CHUNK)], sem_ob
                ).wait()

            select_store(lcb, rows_b, trans_b, sem_ob)

        pltpu.make_async_copy(
            trans_a, out_hbm.at[pl.ds(0, dim), pl.ds(0, _CHUNK)], sem_oa
        ).wait()
        pltpu.make_async_copy(
            trans_b, out_hbm.at[pl.ds(0, dim), pl.ds(0, _CHUNK)], sem_ob
        ).wait()

    out = gather_kernel(table4, idx)
    return out.reshape(seq, dim, batch).transpose(2, 0, 1)


# speedup vs baseline: 1.3695x; 1.0036x over previous
import dataclasses

import jax
import jax.numpy as jnp
from jax import lax
from jax.experimental import pallas as pl
from jax.experimental.pallas import tpu as pltpu
from jax.experimental.pallas import tpu_sc as plsc

_NUM_CORES = 2
_NUM_SUBCORES = 16
_NUM_WORKERS = _NUM_CORES * _NUM_SUBCORES
_CHUNK = 128
_LANES = 16


def kernel(inputs, lookup_table):
    batch, seq = inputs.shape
    vocab, dim = lookup_table.shape
    n = batch * seq
    rows_per_tile = 128 // dim
    n_chunks = n // _CHUNK
    chunks_per_worker = n_chunks // _NUM_WORKERS
    chunks_per_col = batch // _CHUNK
    assert n % (_NUM_WORKERS * _CHUNK) == 0 and batch % _CHUNK == 0
    assert (chunks_per_worker - 2) % 3 == 0

    mesh = plsc.VectorSubcoreMesh(core_axis_name="c", subcore_axis_name="s")
    idx = inputs.T.reshape(n).astype(jnp.int32)
    table4 = lookup_table.reshape(vocab // rows_per_tile, 128)

    cp = pltpu.CompilerParams()
    if "needs_layout_passes" in pltpu.CompilerParams.__dataclass_fields__:
        cp = dataclasses.replace(cp, needs_layout_passes=False)

    @pl.kernel(
        out_type=jax.ShapeDtypeStruct((seq * dim, batch), jnp.float32),
        mesh=mesh,
        compiler_params=cp,
        scratch_types=[
            pltpu.VMEM((chunks_per_worker * _CHUNK,), jnp.int32),
            pltpu.VMEM((_CHUNK,), jnp.int32),
            pltpu.VMEM((_CHUNK,), jnp.int32),
            pltpu.VMEM((_CHUNK,), jnp.int32),
            pltpu.VMEM((_CHUNK, 128), jnp.float32),
            pltpu.VMEM((_CHUNK, 128), jnp.float32),
            pltpu.VMEM((_CHUNK, 128), jnp.float32),
            pltpu.VMEM((dim, _CHUNK), jnp.float32),
            pltpu.VMEM((dim, _CHUNK), jnp.float32),
            pltpu.VMEM((dim, _CHUNK), jnp.float32),
            pltpu.SemaphoreType.DMA,
            pltpu.SemaphoreType.DMA,
            pltpu.SemaphoreType.DMA,
            pltpu.SemaphoreType.DMA,
            pltpu.SemaphoreType.DMA,
            pltpu.SemaphoreType.DMA,
        ],
    )
    def gather_kernel(
        table_hbm,
        idx_hbm,
        out_hbm,
        idx_all,
        r_a,
        r_b,
        r_c,
        rows_a,
        rows_b,
        rows_c,
        trans_a,
        trans_b,
        trans_c,
        sem_ga,
        sem_gb,
        sem_gc,
        sem_oa,
        sem_ob,
        sem_oc,
    ):
        wid = lax.axis_index("s") * _NUM_CORES + lax.axis_index("c")
        first = wid * chunks_per_worker
        iota = lax.iota(jnp.int32, _LANES)
        groups = _CHUNK // _LANES

        base0 = pl.multiple_of(first * _CHUNK, _CHUNK)
        pltpu.sync_copy(
            idx_hbm.at[pl.ds(base0, chunks_per_worker * _CHUNK)], idx_all
        )

        def fetch(lc, r_v, rows_v, sem_g):
            off = lc * _CHUNK
            for g in range(groups):
                v = idx_all[pl.ds(off + g * _LANES, _LANES)]
                r_v[pl.ds(g * _LANES, _LANES)] = lax.shift_right_logical(v, 2)
            return pltpu.async_copy(table_hbm.at[r_v], rows_v, sem_g)

        def select_store(lc, rows_v, trans_v, sem_o):
            off = lc * _CHUNK
            cols = []
            rows16 = []
            for g in range(groups):
                v = idx_all[pl.ds(off + g * _LANES, _LANES)]
                cols.append((v & (rows_per_tile - 1)) * dim)
                rows16.append(g * _LANES + iota)
            for e in range(dim):
                for g in range(groups):
                    val = plsc.load_gather(rows_v, [rows16[g], cols[g]])
                    trans_v[e, pl.ds(g * _LANES, _LANES)] = val
                    if e + 1 < dim:
                        cols[g] = cols[g] + 1
            ch = first + lc
            t = ch // chunks_per_col
            b0 = (ch % chunks_per_col) * _CHUNK
            r0 = pl.multiple_of(t * dim, 8)
            return pltpu.async_copy(
                trans_v, out_hbm.at[pl.ds(r0, dim), pl.ds(b0, _CHUNK)], sem_o
            )

        def wait_gather(rows_v, sem_g):
            pltpu.make_async_copy(
                table_hbm.at[pl.ds(0, _CHUNK)], rows_v, sem_g
            ).wait()

        def wait_out(trans_v, sem_o):
            pltpu.make_async_copy(
                trans_v, out_hbm.at[pl.ds(0, dim), pl.ds(0, _CHUNK)], sem_o
            ).wait()

        rounds = (chunks_per_worker - 2) // 3

        fetch(0, r_a, rows_a, sem_ga)
        fetch(1, r_b, rows_b, sem_gb)

        @pl.loop(0, rounds)
        def _(k):
            base = 3 * k
            fetch(base + 2, r_c, rows_c, sem_gc)
            wait_gather(rows_a, sem_ga)

            @pl.when(k > 0)
            def _():
                wait_out(trans_a, sem_oa)

            select_store(base, rows_a, trans_a, sem_oa)

            @pl.when(base + 3 < chunks_per_worker)
            def _():
                fetch(base + 3, r_a, rows_a, sem_ga)

            wait_gather(rows_b, sem_gb)

            @pl.when(k > 0)
            def _():
                wait_out(trans_b, sem_ob)

            select_store(base + 1, rows_b, trans_b, sem_ob)

            @pl.when(base + 4 < chunks_per_worker)
            def _():
                fetch(base + 4, r_b, rows_b, sem_gb)

            wait_gather(rows_c, sem_gc)

            @pl.when(k > 0)
            def _():
                wait_out(trans_c, sem_oc)

            select_store(base + 2, rows_c, trans_c, sem_oc)

        wait_gather(rows_a, sem_ga)
        wait_out(trans_a, sem_oa)
        select_store(chunks_per_worker - 2, rows_a, trans_a, sem_oa)
        wait_gather(rows_b, sem_gb)
        wait_out(trans_b, sem_ob)
        select_store(chunks_per_worker - 1, rows_b, trans_b, sem_ob)
        wait_out(trans_a, sem_oa)
        wait_out(trans_b, sem_ob)
        wait_out(trans_c, sem_oc)

    out = gather_kernel(table4, idx)
    return out.reshape(seq, dim, batch).transpose(2, 0, 1)
